# baseline (speedup 1.0000x reference)
import functools

import jax
import jax.numpy as jnp
from jax.experimental import pallas as pl
from jax.experimental.pallas import tpu as pltpu

_NUM_CLASSES = 19
_HIDDEN = 256
_STEPS = 2
_NUM_POINTS = 8192


def _mlp_body(fine_ref, coarse_ref, w1_ref, b1_ref, w2_ref, b2_ref,
              w3_ref, b3_ref, wp_ref, bp_ref, out_ref):
    fine = fine_ref[0]
    coarse = coarse_ref[0]
    x = jnp.concatenate([fine, coarse], axis=0)
    for w_ref, b_ref in ((w1_ref, b1_ref), (w2_ref, b2_ref), (w3_ref, b3_ref)):
        h = jnp.maximum(
            jnp.dot(w_ref[...], x, preferred_element_type=jnp.float32)
            + b_ref[...], 0.0)
        x = jnp.concatenate([h, coarse], axis=0)
    out_ref[0] = (jnp.dot(wp_ref[...], x, preferred_element_type=jnp.float32)
                  + bp_ref[...])


def _point_head(fine, coarse, w1, b1, w2, b2, w3, b3, wp, bp):
    n, _, r = fine.shape
    rb = 2048
    grid = (n, r // rb)
    wspec = pl.BlockSpec(index_map=lambda i, j: (0, 0))
    return pl.pallas_call(
        _mlp_body,
        grid=grid,
        in_specs=[
            pl.BlockSpec((1, 256, rb), lambda i, j: (i, 0, j)),
            pl.BlockSpec((1, _NUM_CLASSES, rb), lambda i, j: (i, 0, j)),
            wspec, wspec, wspec, wspec, wspec, wspec, wspec, wspec,
        ],
        out_specs=pl.BlockSpec((1, _NUM_CLASSES, rb), lambda i, j: (i, 0, j)),
        out_shape=jax.ShapeDtypeStruct((n, _NUM_CLASSES, r), jnp.float32),
    )(fine, coarse, w1, b1[:, None], w2, b2[:, None], w3, b3[:, None],
      wp, bp[:, None])


def _uncertainty(logits):
    t = jnp.transpose(logits, (0, 2, 3, 1))
    vals, _ = jax.lax.top_k(t, 2)
    return (vals[..., 1] - vals[..., 0])[:, None, :, :]


def _grid_points(unc, num_points):
    n = unc.shape[0]
    h, w = unc.shape[2], unc.shape[3]
    r = min(h * w, num_points)
    flat = unc.reshape(n, h * w)
    _, idx = jax.lax.top_k(flat, r)
    xs = (0.5 / w) + (idx % w).astype(jnp.float32) * (1.0 / w)
    ys = (0.5 / h) + (idx // w).astype(jnp.float32) * (1.0 / h)
    return idx, jnp.stack([xs, ys], axis=-1)


def _point_sample(feat, coords):
    n, c, h, w = feat.shape
    x = coords[..., 0] * w - 0.5
    y = coords[..., 1] * h - 0.5
    x0 = jnp.floor(x)
    y0 = jnp.floor(y)

    def corner(xi, yi, wgt):
        valid = ((xi >= 0) & (xi <= w - 1) & (yi >= 0) & (yi <= h - 1)).astype(feat.dtype)
        xc = jnp.clip(xi, 0, w - 1).astype(jnp.int32)
        yc = jnp.clip(yi, 0, h - 1).astype(jnp.int32)
        vals = jax.vmap(lambda f, yy, xx: f[:, yy, xx])(feat, yc, xc)
        return vals * (wgt * valid)[:, None, :]

    wx1 = x - x0
    wx0 = 1.0 - wx1
    wy1 = y - y0
    wy0 = 1.0 - wy1
    return (corner(x0, y0, wx0 * wy0) + corner(x0 + 1.0, y0, wx1 * wy0)
            + corner(x0, y0 + 1.0, wx0 * wy1) + corner(x0 + 1.0, y0 + 1.0, wx1 * wy1))


def kernel(features, w_coarse, b_coarse, w_fine, b_fine, w1, b1, w2, b2, w3, b3, wp, bp):
    coarse_logits = (jnp.einsum('nchw,kc->nkhw', features, w_coarse)
                     + b_coarse[None, :, None, None])
    low_level_feat = (jnp.einsum('nchw,kc->nkhw', features, w_fine)
                      + b_fine[None, :, None, None])
    sem = coarse_logits
    for _ in range(_STEPS):
        n, c, h, w = sem.shape
        sem = jax.image.resize(sem, (n, c, h * 2, w * 2), method='bilinear')
        unc = _uncertainty(sem)
        idx, coords = _grid_points(unc, _NUM_POINTS)
        fine = _point_sample(low_level_feat, coords)
        coarse_f = _point_sample(coarse_logits, coords)
        plog = _point_head(fine, coarse_f, w1, b1, w2, b2, w3, b3, wp, bp)
        hn, wn = h * 2, w * 2
        flat = sem.reshape(n, c, hn * wn)
        flat = jax.vmap(lambda f, i, v: f.at[:, i].set(v))(flat, idx, plog)
        sem = flat.reshape(n, c, hn, wn)
    return sem


# speedup vs baseline: 1.3642x; 1.3642x over previous
import functools

import numpy as np

import jax
import jax.numpy as jnp
from jax.experimental import pallas as pl
from jax.experimental.pallas import tpu as pltpu

_C = 19
_F = 256
_STEPS = 2
_K = 8192


def _upsample_matrix(h):
    h2 = 2 * h
    fx = (np.arange(h2) + 0.5) / 2.0 - 0.5
    a = np.arange(-1, h + 1)
    w = np.maximum(0.0, 1.0 - np.abs(fx[:, None] - a[None, :]))
    ac = np.clip(a, 0, h - 1)
    u = np.zeros((h2, h), np.float32)
    for r in range(h2):
        for j in range(len(a)):
            u[r, ac[j]] += w[r, j]
    return u


def _up_unc_body(sem_ref, uh_ref, uwt_ref, out_ref, unc_ref):
    h2 = unc_ref.shape[1]
    w2 = unc_ref.shape[2]
    m1 = jnp.full((h2, w2), -jnp.inf, jnp.float32)
    m2 = jnp.full((h2, w2), -jnp.inf, jnp.float32)
    for c in range(_C):
        t = jnp.dot(uh_ref[...], sem_ref[0, c], preferred_element_type=jnp.float32)
        o = jnp.dot(t, uwt_ref[...], preferred_element_type=jnp.float32)
        out_ref[0, c] = o
        m2 = jnp.maximum(m2, jnp.minimum(m1, o))
        m1 = jnp.maximum(m1, o)
    unc_ref[0] = m2 - m1


def _upsample_unc(sem, uh, uwt):
    n, c, h, w = sem.shape
    h2, w2 = 2 * h, 2 * w
    return pl.pallas_call(
        _up_unc_body,
        grid=(n,),
        in_specs=[
            pl.BlockSpec((1, c, h, w), lambda i: (i, 0, 0, 0)),
            pl.BlockSpec(index_map=lambda i: (0, 0)),
            pl.BlockSpec(index_map=lambda i: (0, 0)),
        ],
        out_specs=[
            pl.BlockSpec((1, c, h2, w2), lambda i: (i, 0, 0, 0)),
            pl.BlockSpec((1, h2, w2), lambda i: (i, 0, 0)),
        ],
        out_shape=[
            jax.ShapeDtypeStruct((n, c, h2, w2), jnp.float32),
            jax.ShapeDtypeStruct((n, h2, w2), jnp.float32),
        ],
    )(sem, uh, uwt)


def _mlp_body(fineg_ref, coarse_ref, sw_ref, w1c_ref, b1_ref, wfb_ref,
              w2a_ref, w2c_ref, b2_ref, w3a_ref, w3c_ref, b3_ref,
              wpa_ref, wpc_ref, bp_ref, out_ref):
    coarse = coarse_ref[0]
    x = jnp.maximum(
        fineg_ref[0]
        + jnp.dot(w1c_ref[...], coarse, preferred_element_type=jnp.float32)
        + b1_ref[...] + wfb_ref[...] * sw_ref[0], 0.0)
    for wa, wc, b in ((w2a_ref, w2c_ref, b2_ref), (w3a_ref, w3c_ref, b3_ref)):
        x = jnp.maximum(
            jnp.dot(wa[...], x, preferred_element_type=jnp.float32)
            + jnp.dot(wc[...], coarse, preferred_element_type=jnp.float32)
            + b[...], 0.0)
    out_ref[0] = (jnp.dot(wpa_ref[...], x, preferred_element_type=jnp.float32)
                  + jnp.dot(wpc_ref[...], coarse, preferred_element_type=jnp.float32)
                  + bp_ref[...])


def _point_head(fine_g, coarse_f, swsum, w1c, b1, wfb, w2a, w2c, b2,
                w3a, w3c, b3, wpa, wpc, bp):
    n, _, r = fine_g.shape
    rb = 2048
    grid = (n, r // rb)
    wspec = pl.BlockSpec(index_map=lambda i, j: (0, 0))
    return pl.pallas_call(
        _mlp_body,
        grid=grid,
        in_specs=[
            pl.BlockSpec((1, _F, rb), lambda i, j: (i, 0, j)),
            pl.BlockSpec((1, _C, rb), lambda i, j: (i, 0, j)),
            pl.BlockSpec((1, 1, rb), lambda i, j: (i, 0, j)),
        ] + [wspec] * 12,
        out_specs=pl.BlockSpec((1, _C, rb), lambda i, j: (i, 0, j)),
        out_shape=jax.ShapeDtypeStruct((n, _C, r), jnp.float32),
    )(fine_g, coarse_f, swsum[:, None, :], w1c, b1[:, None], wfb[:, None],
      w2a, w2c, b2[:, None], w3a, w3c, b3[:, None], wpa, wpc, bp[:, None])


def _corners(idx, w2, ratio, feat_w):
    ix = idx % w2
    iy = idx // w2
    sc = 2 * ratio
    fxn = 2 * ix - (ratio - 1)
    fyn = 2 * iy - (ratio - 1)
    x0 = fxn // sc
    y0 = fyn // sc
    wx1 = (fxn - x0 * sc).astype(jnp.float32) / sc
    wy1 = (fyn - y0 * sc).astype(jnp.float32) / sc
    return x0, y0, wx1, wy1


def _sample_and_swsum(feat, x0, y0, wx1, wy1):
    n, c, h, w = feat.shape
    acc = None
    sw = None
    for dy in (0, 1):
        for dx in (0, 1):
            xi = x0 + dx
            yi = y0 + dy
            valid = ((xi >= 0) & (xi <= w - 1) & (yi >= 0) & (yi <= h - 1))
            wgt = (jnp.where(dx == 1, wx1, 1.0 - wx1)
                   * jnp.where(dy == 1, wy1, 1.0 - wy1)
                   * valid.astype(jnp.float32))
            xc = jnp.clip(xi, 0, w - 1)
            yc = jnp.clip(yi, 0, h - 1)
            vals = jax.vmap(lambda f, yy, xx: f[:, yy, xx])(feat, yc, xc)
            term = vals * wgt[:, None, :]
            acc = term if acc is None else acc + term
            sw = wgt if sw is None else sw + wgt
    return acc, sw


def kernel(features, w_coarse, b_coarse, w_fine, b_fine, w1, b1, w2, b2, w3, b3, wp, bp):
    n, cf, h0, w0 = features.shape

    w1f = w1[:, :_F]
    w1c = w1[:, _F:]
    w_eff = w1f @ w_fine
    wfb = w1f @ b_fine
    w2a, w2c = w2[:, :_F], w2[:, _F:]
    w3a, w3c = w3[:, :_F], w3[:, _F:]
    wpa, wpc = wp[:, :_F], wp[:, _F:]

    coarse_logits = (jnp.einsum('nchw,kc->nkhw', features, w_coarse)
                     + b_coarse[None, :, None, None])
    g = jnp.einsum('nchw,kc->nkhw', features, w_eff)

    uh1 = jnp.asarray(_upsample_matrix(h0))
    uw1t = jnp.asarray(_upsample_matrix(w0).T)
    uh2 = jnp.asarray(_upsample_matrix(2 * h0))
    uw2t = jnp.asarray(_upsample_matrix(2 * w0).T)

    sem = coarse_logits
    for step in range(_STEPS):
        uh, uwt = (uh1, uw1t) if step == 0 else (uh2, uw2t)
        sem_up, unc = _upsample_unc(sem, uh, uwt)
        nn, cc, h2, w2 = sem_up.shape
        flat_unc = unc.reshape(n, h2 * w2)
        _, idx = jax.lax.top_k(flat_unc, _K)
        ratio = (2 * h0 * 2 ** step) // h0
        x0, y0, wx1, wy1 = _corners(idx, w2, ratio, w0)
        fine_g, swsum = _sample_and_swsum(g, x0, y0, wx1, wy1)
        coarse_f, _ = _sample_and_swsum(coarse_logits, x0, y0, wx1, wy1)
        plog = _point_head(fine_g, coarse_f, swsum, w1c, b1, wfb,
                           w2a, w2c, b2, w3a, w3c, b3, wpa, wpc, bp)
        flat = sem_up.reshape(n, cc, h2 * w2)
        flat = jax.vmap(lambda f, i, v: f.at[:, i].set(v))(flat, idx, plog)
        sem = flat.reshape(n, cc, h2, w2)
    return sem
